# baseline (speedup 1.0000x reference)
import functools
import math

import jax
import jax.numpy as jnp
import numpy as np
from jax import lax
from jax.experimental import pallas as pl
from jax.experimental.pallas import tpu as pltpu
from jax.experimental.pallas import tpu_sc as plsc

SEQ = 2048
DIM = 1024
NUM_HEADS = 16
HEAD_DIM = 64
KNBR = 32
SCALE = 1.0 / math.sqrt(HEAD_DIM)

_J = np.arange(DIM)
PERM = np.asarray((_J % NUM_HEADS) * HEAD_DIM + _J // NUM_HEADS, dtype=np.int32)

_W = np.arange(DIM // 2)
_DPW = _W // 16
_HW = _W % 16
PERMK = np.concatenate([
    _HW * HEAD_DIM + 2 * _DPW,
    _HW * HEAD_DIM + 2 * _DPW + 1]).astype(np.int32)
_WPR = DIM // 2


def _mm_kernel(a_ref, w_ref, b_ref, o_ref):
    acc = lax.dot_general(
        a_ref[...], w_ref[...],
        dimension_numbers=(((1,), (1,)), ((), ())),
        preferred_element_type=jnp.float32,
    )
    o_ref[...] = (acc + b_ref[0, :][None, :]).astype(o_ref.dtype)


def _matmul(a, w, b, bm=512, bn=512, out_dtype=jnp.float32):
    a = a.astype(jnp.bfloat16)
    w = w.astype(jnp.bfloat16)
    m, k = a.shape
    n = w.shape[0]
    b2 = b.reshape(1, n)
    return pl.pallas_call(
        _mm_kernel,
        grid=(m // bm, n // bn),
        in_specs=[
            pl.BlockSpec((bm, k), lambda i, j: (i, 0)),
            pl.BlockSpec((bn, k), lambda i, j: (j, 0)),
            pl.BlockSpec((1, bn), lambda i, j: (0, j)),
        ],
        out_specs=pl.BlockSpec((bm, bn), lambda i, j: (i, j)),
        out_shape=jax.ShapeDtypeStruct((m, n), out_dtype),
    )(a, w, b2)


def _mm_pack_kernel(a_ref, w_ref, b_ref, o_ref):
    acc = lax.dot_general(
        a_ref[...], w_ref[...],
        dimension_numbers=(((1,), (1,)), ((), ())),
        preferred_element_type=jnp.float32,
    )
    acc = acc + b_ref[0, :][None, :]
    half = acc.shape[1] // 2
    lo = lax.bitcast_convert_type(
        acc[:, :half].astype(jnp.bfloat16), jnp.uint16).astype(jnp.int32)
    hi = lax.bitcast_convert_type(
        acc[:, half:].astype(jnp.bfloat16), jnp.uint16).astype(jnp.int32)
    o_ref[...] = jnp.bitwise_or(lo, lax.shift_left(hi, 16))


def _pack_half(sub):
    half = sub.shape[1] // 2
    lo = lax.bitcast_convert_type(
        sub[:, :half].astype(jnp.bfloat16), jnp.uint16).astype(jnp.int32)
    hi = lax.bitcast_convert_type(
        sub[:, half:].astype(jnp.bfloat16), jnp.uint16).astype(jnp.int32)
    return jnp.bitwise_or(lo, lax.shift_left(hi, 16))


def _mm_qkv_kernel(a_ref, w_ref, b_ref, q_ref, kv_ref):
    acc = lax.dot_general(
        a_ref[...], w_ref[...],
        dimension_numbers=(((1,), (1,)), ((), ())),
        preferred_element_type=jnp.float32,
    )
    acc = acc + b_ref[0, :][None, :]
    q_ref[...] = acc[:, :DIM]
    kv_ref[:, :DIM // 2] = _pack_half(acc[:, DIM:2 * DIM])
    kv_ref[:, DIM // 2:] = _pack_half(acc[:, 2 * DIM:])


def _matmul_qkv(a, w, b, bm=512):
    a = a.astype(jnp.bfloat16)
    w = w.astype(jnp.bfloat16)
    m, k = a.shape
    n = w.shape[0]
    b2 = b.reshape(1, n)
    return pl.pallas_call(
        _mm_qkv_kernel,
        grid=(m // bm,),
        in_specs=[
            pl.BlockSpec((bm, k), lambda i: (i, 0)),
            pl.BlockSpec((n, k), lambda i: (0, 0)),
            pl.BlockSpec((1, n), lambda i: (0, 0)),
        ],
        out_specs=[
            pl.BlockSpec((bm, DIM), lambda i: (i, 0)),
            pl.BlockSpec((bm, DIM), lambda i: (i, 0)),
        ],
        out_shape=[
            jax.ShapeDtypeStruct((m, DIM), jnp.float32),
            jax.ShapeDtypeStruct((m, DIM), jnp.int32),
        ],
    )(a, w, b2)


def _matmul_packed(a, w, b, bm=512):
    a = a.astype(jnp.bfloat16)
    w = w.astype(jnp.bfloat16)
    m, k = a.shape
    n = w.shape[0]
    b2 = b.reshape(1, n)
    return pl.pallas_call(
        _mm_pack_kernel,
        grid=(m // bm,),
        in_specs=[
            pl.BlockSpec((bm, k), lambda i: (i, 0)),
            pl.BlockSpec((n, k), lambda i: (0, 0)),
            pl.BlockSpec((1, n), lambda i: (0, 0)),
        ],
        out_specs=pl.BlockSpec((bm, n // 2), lambda i: (i, 0)),
        out_shape=jax.ShapeDtypeStruct((m, n // 2), jnp.int32),
    )(a, w, b2)


_NC, _NS = 2, 16
_NW = _NC * _NS
_QPW = SEQ // _NW
_JT = 16
_PT = 8
_NDP = HEAD_DIM // 2


def _bf16_pair(w):
    lo = lax.bitcast_convert_type(jnp.left_shift(w, 16), jnp.float32)
    hi = lax.bitcast_convert_type(w, jnp.float32)
    return lo, hi


def _attn_body(q_hbm, kv_hbm, routes_hbm, out_hbm,
               idx0, idx1, kbuf0, kbuf1, qbuf0, qbuf1, sbuf, orow,
               semk):
    wid = lax.axis_index("s") * _NC + lax.axis_index("c")
    base = wid * _QPW

    idxs = (idx0, idx1)
    kbufs = (kbuf0, kbuf1)
    qbufs = (qbuf0, qbuf1)

    pltpu.sync_copy(routes_hbm.at[base], idx0)
    pltpu.async_copy(kv_hbm.at[idx0], kbuf0, semk)
    pltpu.async_copy(q_hbm.at[base], qbuf0, semk)

    def one_query(slot, i, inext):
        idxc, kb, qb = idxs[slot], kbufs[slot], qbufs[slot]
        idxn, kbn, qbn = idxs[1 - slot], kbufs[1 - slot], qbufs[1 - slot]

        pltpu.make_async_copy(kv_hbm.at[idxc], kb, semk).wait()
        pltpu.make_async_copy(q_hbm.at[i], qb, semk).wait()

        pltpu.sync_copy(routes_hbm.at[inext], idxn)
        pltpu.async_copy(kv_hbm.at[idxn], kbn, semk)
        pltpu.async_copy(q_hbm.at[inext], qbn, semk)

        for jt in range(KNBR // _JT):
            def dot_dp(dp, accs, jt=jt):
                q0 = qb[pl.ds(dp * 32, 16)]
                q1 = qb[pl.ds(dp * 32 + 16, 16)]
                out = []
                for u in range(_JT):
                    a, bb = _bf16_pair(kb[jt * _JT + u, pl.ds(dp * 16, 16)])
                    out.append(accs[u] + q0 * a + q1 * bb)
                return tuple(out)

            accs = lax.fori_loop(
                0, _NDP, dot_dp,
                tuple(jnp.zeros(16, jnp.float32) for _ in range(_JT)),
                unroll=2)
            for u in range(_JT):
                sbuf[jt * _JT + u, :] = accs[u] * SCALE

        def max_j(j, m):
            return jnp.maximum(m, sbuf[j, :])

        m = lax.fori_loop(0, KNBR, max_j,
                          jnp.full((16,), -jnp.inf, jnp.float32), unroll=2)

        def exp_j(j, s):
            e = jnp.exp(sbuf[j, :] - m)
            sbuf[j, :] = e
            return s + e

        s = lax.fori_loop(0, KNBR, exp_j, jnp.zeros(16, jnp.float32), unroll=2)
        r = 1.0 / s

        for pt in range(_NDP // _PT):
            def acc_j(j, accs, pt=pt):
                wv = sbuf[j, :]
                out = list(accs)
                for t in range(_PT):
                    a, bb = _bf16_pair(
                        kb[j, pl.ds(_WPR + (pt * _PT + t) * 16, 16)])
                    out[2 * t] = out[2 * t] + wv * a
                    out[2 * t + 1] = out[2 * t + 1] + wv * bb
                return tuple(out)

            accs = lax.fori_loop(
                0, KNBR, acc_j,
                tuple(jnp.zeros(16, jnp.float32) for _ in range(2 * _PT)),
                unroll=2)
            for t in range(_PT):
                orow[pl.ds((pt * _PT + t) * 32, 16)] = accs[2 * t] * r
                orow[pl.ds((pt * _PT + t) * 32 + 16, 16)] = accs[2 * t + 1] * r

        pltpu.sync_copy(orow, out_hbm.at[i])

    def pair(h, _):
        i0 = base + 2 * h
        one_query(0, i0, i0 + 1)
        inext = jnp.where(2 * h + 2 < _QPW, i0 + 2, base)
        one_query(1, i0 + 1, inext)
        return 0

    lax.fori_loop(0, _QPW // 2, pair, 0)

    pltpu.make_async_copy(kv_hbm.at[idx0], kbuf0, semk).wait()
    pltpu.make_async_copy(q_hbm.at[base], qbuf0, semk).wait()


def _sc_attention(q, kv, routes):
    attn_fn = pl.kernel(
        _attn_body,
        mesh=plsc.VectorSubcoreMesh(core_axis_name="c", subcore_axis_name="s"),
        out_type=jax.ShapeDtypeStruct((SEQ, DIM), jnp.float32),
        scratch_types=[
            pltpu.VMEM((KNBR,), jnp.int32),
            pltpu.VMEM((KNBR,), jnp.int32),
            pltpu.VMEM((KNBR, DIM), jnp.int32),
            pltpu.VMEM((KNBR, DIM), jnp.int32),
            pltpu.VMEM((DIM,), jnp.float32),
            pltpu.VMEM((DIM,), jnp.float32),
            pltpu.VMEM((KNBR, 16), jnp.float32),
            pltpu.VMEM((DIM,), jnp.float32),
            pltpu.SemaphoreType.DMA,
        ],
    )
    return attn_fn(q, kv, routes)


def kernel(x, W_qkv, b_qkv, W_out, b_out, routes):
    xs = x.reshape(SEQ, DIM)
    rows = np.concatenate([PERM, DIM + PERMK, 2 * DIM + PERMK])
    wqkv = jnp.take(W_qkv, rows, axis=0)
    bqkv = jnp.take(b_qkv, rows)

    q, kv = _matmul_qkv(xs, wqkv, bqkv)
    attn = _sc_attention(q, kv, routes)
    out = _matmul(attn, jnp.take(W_out, PERM, axis=1), b_out)
    return out.reshape(1, SEQ, DIM)


# speedup vs baseline: 1.0774x; 1.0774x over previous
import functools
import math

import jax
import jax.numpy as jnp
import numpy as np
from jax import lax
from jax.experimental import pallas as pl
from jax.experimental.pallas import tpu as pltpu
from jax.experimental.pallas import tpu_sc as plsc

SEQ = 2048
DIM = 1024
NUM_HEADS = 16
HEAD_DIM = 64
KNBR = 32
SCALE = 1.0 / math.sqrt(HEAD_DIM)

_J = np.arange(DIM)
PERM = np.asarray((_J % NUM_HEADS) * HEAD_DIM + _J // NUM_HEADS, dtype=np.int32)

_W = np.arange(DIM // 2)
_DPW = _W // 16
_HW = _W % 16
PERMK = np.concatenate([
    _HW * HEAD_DIM + 2 * _DPW,
    _HW * HEAD_DIM + 2 * _DPW + 1]).astype(np.int32)
_WPR = DIM // 2


def _mm_kernel(a_ref, w_ref, b_ref, o_ref):
    acc = lax.dot_general(
        a_ref[...], w_ref[...],
        dimension_numbers=(((1,), (1,)), ((), ())),
        preferred_element_type=jnp.float32,
    )
    o_ref[...] = (acc + b_ref[0, :][None, :]).astype(o_ref.dtype)


def _matmul(a, w, b, bm=512, bn=512, out_dtype=jnp.float32):
    a = a.astype(jnp.bfloat16)
    w = w.astype(jnp.bfloat16)
    m, k = a.shape
    n = w.shape[0]
    b2 = b.reshape(1, n)
    return pl.pallas_call(
        _mm_kernel,
        grid=(m // bm, n // bn),
        in_specs=[
            pl.BlockSpec((bm, k), lambda i, j: (i, 0)),
            pl.BlockSpec((bn, k), lambda i, j: (j, 0)),
            pl.BlockSpec((1, bn), lambda i, j: (0, j)),
        ],
        out_specs=pl.BlockSpec((bm, bn), lambda i, j: (i, j)),
        out_shape=jax.ShapeDtypeStruct((m, n), out_dtype),
    )(a, w, b2)


def _mm_pack_kernel(a_ref, w_ref, b_ref, o_ref):
    acc = lax.dot_general(
        a_ref[...], w_ref[...],
        dimension_numbers=(((1,), (1,)), ((), ())),
        preferred_element_type=jnp.float32,
    )
    acc = acc + b_ref[0, :][None, :]
    half = acc.shape[1] // 2
    lo = lax.bitcast_convert_type(
        acc[:, :half].astype(jnp.bfloat16), jnp.uint16).astype(jnp.int32)
    hi = lax.bitcast_convert_type(
        acc[:, half:].astype(jnp.bfloat16), jnp.uint16).astype(jnp.int32)
    o_ref[...] = jnp.bitwise_or(lo, lax.shift_left(hi, 16))


def _pack_half(sub):
    half = sub.shape[1] // 2
    lo = lax.bitcast_convert_type(
        sub[:, :half].astype(jnp.bfloat16), jnp.uint16).astype(jnp.int32)
    hi = lax.bitcast_convert_type(
        sub[:, half:].astype(jnp.bfloat16), jnp.uint16).astype(jnp.int32)
    return jnp.bitwise_or(lo, lax.shift_left(hi, 16))


def _mm_qkv_kernel(a_ref, w_ref, b_ref, q_ref, kv_ref):
    acc = lax.dot_general(
        a_ref[...], w_ref[...],
        dimension_numbers=(((1,), (1,)), ((), ())),
        preferred_element_type=jnp.float32,
    )
    acc = acc + b_ref[0, :][None, :]
    q_ref[...] = acc[:, :DIM]
    kv_ref[:, :DIM // 2] = _pack_half(acc[:, DIM:2 * DIM])
    kv_ref[:, DIM // 2:] = _pack_half(acc[:, 2 * DIM:])


def _matmul_qkv(a, w, b, bm=512):
    a = a.astype(jnp.bfloat16)
    w = w.astype(jnp.bfloat16)
    m, k = a.shape
    n = w.shape[0]
    b2 = b.reshape(1, n)
    return pl.pallas_call(
        _mm_qkv_kernel,
        grid=(m // bm,),
        in_specs=[
            pl.BlockSpec((bm, k), lambda i: (i, 0)),
            pl.BlockSpec((n, k), lambda i: (0, 0)),
            pl.BlockSpec((1, n), lambda i: (0, 0)),
        ],
        out_specs=[
            pl.BlockSpec((bm, DIM), lambda i: (i, 0)),
            pl.BlockSpec((bm, DIM), lambda i: (i, 0)),
        ],
        out_shape=[
            jax.ShapeDtypeStruct((m, DIM), jnp.float32),
            jax.ShapeDtypeStruct((m, DIM), jnp.int32),
        ],
    )(a, w, b2)


def _matmul_packed(a, w, b, bm=512):
    a = a.astype(jnp.bfloat16)
    w = w.astype(jnp.bfloat16)
    m, k = a.shape
    n = w.shape[0]
    b2 = b.reshape(1, n)
    return pl.pallas_call(
        _mm_pack_kernel,
        grid=(m // bm,),
        in_specs=[
            pl.BlockSpec((bm, k), lambda i: (i, 0)),
            pl.BlockSpec((n, k), lambda i: (0, 0)),
            pl.BlockSpec((1, n), lambda i: (0, 0)),
        ],
        out_specs=pl.BlockSpec((bm, n // 2), lambda i: (i, 0)),
        out_shape=jax.ShapeDtypeStruct((m, n // 2), jnp.int32),
    )(a, w, b2)


_NC, _NS = 2, 16
_NW = _NC * _NS
_QPW = SEQ // _NW
_JT = 8
_PT = 8
_NDP = HEAD_DIM // 2


def _bf16_pair(w):
    lo = lax.bitcast_convert_type(jnp.left_shift(w, 16), jnp.float32)
    hi = lax.bitcast_convert_type(w, jnp.float32)
    return lo, hi


def _attn_body(q_hbm, kv_hbm, routes_hbm, out_hbm,
               idx0, idx1, kbuf0, kbuf1, qbuf0, qbuf1, sbuf, orow,
               semk):
    wid = lax.axis_index("s") * _NC + lax.axis_index("c")
    base = wid * _QPW

    idxs = (idx0, idx1)
    kbufs = (kbuf0, kbuf1)
    qbufs = (qbuf0, qbuf1)

    pltpu.sync_copy(routes_hbm.at[base], idx0)
    pltpu.async_copy(kv_hbm.at[idx0], kbuf0, semk)
    pltpu.async_copy(q_hbm.at[base], qbuf0, semk)

    def one_query(slot, i, inext):
        idxc, kb, qb = idxs[slot], kbufs[slot], qbufs[slot]
        idxn, kbn, qbn = idxs[1 - slot], kbufs[1 - slot], qbufs[1 - slot]

        pltpu.make_async_copy(kv_hbm.at[idxc], kb, semk).wait()
        pltpu.make_async_copy(q_hbm.at[i], qb, semk).wait()

        pltpu.sync_copy(routes_hbm.at[inext], idxn)
        pltpu.async_copy(kv_hbm.at[idxn], kbn, semk)
        pltpu.async_copy(q_hbm.at[inext], qbn, semk)

        m = jnp.full((16,), -jnp.inf, jnp.float32)
        for jt in range(KNBR // _JT):
            def dot_dp(dp, accs, jt=jt):
                q0 = qb[pl.ds(dp * 32, 16)]
                q1 = qb[pl.ds(dp * 32 + 16, 16)]
                out = []
                for u in range(_JT):
                    a, bb = _bf16_pair(kb[jt * _JT + u, pl.ds(dp * 16, 16)])
                    out.append(accs[u] + q0 * a + q1 * bb)
                return tuple(out)

            accs = lax.fori_loop(
                0, _NDP, dot_dp,
                tuple(jnp.zeros(16, jnp.float32) for _ in range(_JT)),
                unroll=2)
            for u in range(_JT):
                sc = accs[u] * SCALE
                sbuf[jt * _JT + u, :] = sc
                m = jnp.maximum(m, sc)

        def exp_j(j, s):
            e = jnp.exp(sbuf[j, :] - m)
            sbuf[j, :] = e
            return s + e

        s = lax.fori_loop(0, KNBR, exp_j, jnp.zeros(16, jnp.float32), unroll=2)
        r = 1.0 / s

        for pt in range(_NDP // _PT):
            def acc_j(j, accs, pt=pt):
                wv = sbuf[j, :]
                out = list(accs)
                for t in range(_PT):
                    a, bb = _bf16_pair(
                        kb[j, pl.ds(_WPR + (pt * _PT + t) * 16, 16)])
                    out[2 * t] = out[2 * t] + wv * a
                    out[2 * t + 1] = out[2 * t + 1] + wv * bb
                return tuple(out)

            accs = lax.fori_loop(
                0, KNBR, acc_j,
                tuple(jnp.zeros(16, jnp.float32) for _ in range(2 * _PT)),
                unroll=2)
            for t in range(_PT):
                orow[pl.ds((pt * _PT + t) * 32, 16)] = accs[2 * t] * r
                orow[pl.ds((pt * _PT + t) * 32 + 16, 16)] = accs[2 * t + 1] * r

        pltpu.sync_copy(orow, out_hbm.at[i])

    def pair(h, _):
        i0 = base + 2 * h
        one_query(0, i0, i0 + 1)
        inext = jnp.where(2 * h + 2 < _QPW, i0 + 2, base)
        one_query(1, i0 + 1, inext)
        return 0

    lax.fori_loop(0, _QPW // 2, pair, 0)

    pltpu.make_async_copy(kv_hbm.at[idx0], kbuf0, semk).wait()
    pltpu.make_async_copy(q_hbm.at[base], qbuf0, semk).wait()


def _sc_attention(q, kv, routes):
    attn_fn = pl.kernel(
        _attn_body,
        mesh=plsc.VectorSubcoreMesh(core_axis_name="c", subcore_axis_name="s"),
        out_type=jax.ShapeDtypeStruct((SEQ, DIM), jnp.float32),
        scratch_types=[
            pltpu.VMEM((KNBR,), jnp.int32),
            pltpu.VMEM((KNBR,), jnp.int32),
            pltpu.VMEM((KNBR, DIM), jnp.int32),
            pltpu.VMEM((KNBR, DIM), jnp.int32),
            pltpu.VMEM((DIM,), jnp.float32),
            pltpu.VMEM((DIM,), jnp.float32),
            pltpu.VMEM((KNBR, 16), jnp.float32),
            pltpu.VMEM((DIM,), jnp.float32),
            pltpu.SemaphoreType.DMA,
        ],
    )
    return attn_fn(q, kv, routes)


def kernel(x, W_qkv, b_qkv, W_out, b_out, routes):
    xs = x.reshape(SEQ, DIM)
    rows = np.concatenate([PERM, DIM + PERMK, 2 * DIM + PERMK])
    wqkv = jnp.take(W_qkv, rows, axis=0)
    bqkv = jnp.take(b_qkv, rows)

    q, kv = _matmul_qkv(xs, wqkv, bqkv)
    attn = _sc_attention(q, kv, routes)
    out = _matmul(attn, jnp.take(W_out, PERM, axis=1), b_out)
    return out.reshape(1, SEQ, DIM)


# speedup vs baseline: 1.0844x; 1.0065x over previous
import functools
import math

import jax
import jax.numpy as jnp
import numpy as np
from jax import lax
from jax.experimental import pallas as pl
from jax.experimental.pallas import tpu as pltpu
from jax.experimental.pallas import tpu_sc as plsc

SEQ = 2048
DIM = 1024
NUM_HEADS = 16
HEAD_DIM = 64
KNBR = 32
SCALE = 1.0 / math.sqrt(HEAD_DIM)

_J = np.arange(DIM)
PERM = np.asarray((_J % NUM_HEADS) * HEAD_DIM + _J // NUM_HEADS, dtype=np.int32)

_W = np.arange(DIM // 2)
_DPW = _W // 16
_HW = _W % 16
PERMK = np.concatenate([
    _HW * HEAD_DIM + 2 * _DPW,
    _HW * HEAD_DIM + 2 * _DPW + 1]).astype(np.int32)
_WPR = DIM // 2


def _mm_kernel(a_ref, w_ref, b_ref, o_ref):
    acc = lax.dot_general(
        a_ref[...], w_ref[...],
        dimension_numbers=(((1,), (1,)), ((), ())),
        preferred_element_type=jnp.float32,
    )
    o_ref[...] = (acc + b_ref[0, :][None, :]).astype(o_ref.dtype)


def _matmul(a, w, b, bm=512, bn=512, out_dtype=jnp.float32):
    a = a.astype(jnp.bfloat16)
    w = w.astype(jnp.bfloat16)
    m, k = a.shape
    n = w.shape[0]
    b2 = b.reshape(1, n)
    return pl.pallas_call(
        _mm_kernel,
        grid=(m // bm, n // bn),
        in_specs=[
            pl.BlockSpec((bm, k), lambda i, j: (i, 0)),
            pl.BlockSpec((bn, k), lambda i, j: (j, 0)),
            pl.BlockSpec((1, bn), lambda i, j: (0, j)),
        ],
        out_specs=pl.BlockSpec((bm, bn), lambda i, j: (i, j)),
        out_shape=jax.ShapeDtypeStruct((m, n), out_dtype),
    )(a, w, b2)


def _mm_pack_kernel(a_ref, w_ref, b_ref, o_ref):
    acc = lax.dot_general(
        a_ref[...], w_ref[...],
        dimension_numbers=(((1,), (1,)), ((), ())),
        preferred_element_type=jnp.float32,
    )
    acc = acc + b_ref[0, :][None, :]
    half = acc.shape[1] // 2
    lo = lax.bitcast_convert_type(
        acc[:, :half].astype(jnp.bfloat16), jnp.uint16).astype(jnp.int32)
    hi = lax.bitcast_convert_type(
        acc[:, half:].astype(jnp.bfloat16), jnp.uint16).astype(jnp.int32)
    o_ref[...] = jnp.bitwise_or(lo, lax.shift_left(hi, 16))


def _pack_half(sub):
    half = sub.shape[1] // 2
    lo = lax.bitcast_convert_type(
        sub[:, :half].astype(jnp.bfloat16), jnp.uint16).astype(jnp.int32)
    hi = lax.bitcast_convert_type(
        sub[:, half:].astype(jnp.bfloat16), jnp.uint16).astype(jnp.int32)
    return jnp.bitwise_or(lo, lax.shift_left(hi, 16))


def _mm_qkv_kernel(a_ref, w_ref, b_ref, q_ref, kv_ref):
    acc = lax.dot_general(
        a_ref[...], w_ref[...],
        dimension_numbers=(((1,), (1,)), ((), ())),
        preferred_element_type=jnp.float32,
    )
    acc = acc + b_ref[0, :][None, :]
    q_ref[...] = acc[:, :DIM]
    kv_ref[:, :DIM // 2] = _pack_half(acc[:, DIM:2 * DIM])
    kv_ref[:, DIM // 2:] = _pack_half(acc[:, 2 * DIM:])


def _matmul_qkv(a, w, b, bm=512):
    a = a.astype(jnp.bfloat16)
    w = w.astype(jnp.bfloat16)
    m, k = a.shape
    n = w.shape[0]
    b2 = b.reshape(1, n)
    return pl.pallas_call(
        _mm_qkv_kernel,
        grid=(m // bm,),
        in_specs=[
            pl.BlockSpec((bm, k), lambda i: (i, 0)),
            pl.BlockSpec((n, k), lambda i: (0, 0)),
            pl.BlockSpec((1, n), lambda i: (0, 0)),
        ],
        out_specs=[
            pl.BlockSpec((bm, DIM), lambda i: (i, 0)),
            pl.BlockSpec((bm, DIM), lambda i: (i, 0)),
        ],
        out_shape=[
            jax.ShapeDtypeStruct((m, DIM), jnp.float32),
            jax.ShapeDtypeStruct((m, DIM), jnp.int32),
        ],
    )(a, w, b2)


def _matmul_packed(a, w, b, bm=512):
    a = a.astype(jnp.bfloat16)
    w = w.astype(jnp.bfloat16)
    m, k = a.shape
    n = w.shape[0]
    b2 = b.reshape(1, n)
    return pl.pallas_call(
        _mm_pack_kernel,
        grid=(m // bm,),
        in_specs=[
            pl.BlockSpec((bm, k), lambda i: (i, 0)),
            pl.BlockSpec((n, k), lambda i: (0, 0)),
            pl.BlockSpec((1, n), lambda i: (0, 0)),
        ],
        out_specs=pl.BlockSpec((bm, n // 2), lambda i: (i, 0)),
        out_shape=jax.ShapeDtypeStruct((m, n // 2), jnp.int32),
    )(a, w, b2)


_NC, _NS = 2, 16
_NW = _NC * _NS
_QPW = SEQ // _NW
_JT = 8
_PT = 8
_NDP = HEAD_DIM // 2


def _bf16_pair(w):
    lo = lax.bitcast_convert_type(jnp.left_shift(w, 16), jnp.float32)
    hi = lax.bitcast_convert_type(w, jnp.float32)
    return lo, hi


def _attn_body(q_hbm, kv_hbm, routes_hbm, out_hbm,
               idx0, idx1, kbuf0, kbuf1, qbuf0, qbuf1, sbuf, orow,
               semk):
    wid = lax.axis_index("s") * _NC + lax.axis_index("c")
    base = wid * _QPW

    idxs = (idx0, idx1)
    kbufs = (kbuf0, kbuf1)
    qbufs = (qbuf0, qbuf1)

    pltpu.sync_copy(routes_hbm.at[base], idx0)
    pltpu.async_copy(kv_hbm.at[idx0], kbuf0, semk)
    pltpu.async_copy(q_hbm.at[base], qbuf0, semk)

    def one_query(slot, i, inext):
        idxc, kb, qb = idxs[slot], kbufs[slot], qbufs[slot]
        idxn, kbn, qbn = idxs[1 - slot], kbufs[1 - slot], qbufs[1 - slot]

        pltpu.make_async_copy(kv_hbm.at[idxc], kb, semk).wait()
        pltpu.make_async_copy(q_hbm.at[i], qb, semk).wait()

        pltpu.sync_copy(routes_hbm.at[inext], idxn)
        pltpu.async_copy(kv_hbm.at[idxn], kbn, semk)
        pltpu.async_copy(q_hbm.at[inext], qbn, semk)

        m = jnp.full((16,), -jnp.inf, jnp.float32)
        for jt in range(KNBR // _JT):
            def dot_dp(dp, accs, jt=jt):
                q0 = qb[pl.ds(dp * 32, 16)]
                q1 = qb[pl.ds(dp * 32 + 16, 16)]
                out = []
                for u in range(_JT):
                    a, bb = _bf16_pair(kb[jt * _JT + u, pl.ds(dp * 16, 16)])
                    out.append(accs[u] + q0 * a + q1 * bb)
                return tuple(out)

            accs = lax.fori_loop(
                0, _NDP, dot_dp,
                tuple(jnp.zeros(16, jnp.float32) for _ in range(_JT)),
                unroll=2)
            for u in range(_JT):
                sc = accs[u]
                sbuf[jt * _JT + u, :] = sc
                m = jnp.maximum(m, sc)

        def exp_j(j, s):
            e = jnp.exp(sbuf[j, :] - m)
            sbuf[j, :] = e
            return s + e

        s = lax.fori_loop(0, KNBR, exp_j, jnp.zeros(16, jnp.float32), unroll=2)
        r = 1.0 / s

        for pt in range(_NDP // _PT):
            def acc_j(j, accs, pt=pt):
                wv = sbuf[j, :]
                out = list(accs)
                for t in range(_PT):
                    a, bb = _bf16_pair(
                        kb[j, pl.ds(_WPR + (pt * _PT + t) * 16, 16)])
                    out[2 * t] = out[2 * t] + wv * a
                    out[2 * t + 1] = out[2 * t + 1] + wv * bb
                return tuple(out)

            accs = lax.fori_loop(
                0, KNBR, acc_j,
                tuple(jnp.zeros(16, jnp.float32) for _ in range(2 * _PT)),
                unroll=2)
            for t in range(_PT):
                orow[pl.ds((pt * _PT + t) * 32, 16)] = accs[2 * t] * r
                orow[pl.ds((pt * _PT + t) * 32 + 16, 16)] = accs[2 * t + 1] * r

        pltpu.sync_copy(orow, out_hbm.at[i])

    def pair(h, _):
        i0 = base + 2 * h
        one_query(0, i0, i0 + 1)
        inext = jnp.where(2 * h + 2 < _QPW, i0 + 2, base)
        one_query(1, i0 + 1, inext)
        return 0

    lax.fori_loop(0, _QPW // 2, pair, 0)

    pltpu.make_async_copy(kv_hbm.at[idx0], kbuf0, semk).wait()
    pltpu.make_async_copy(q_hbm.at[base], qbuf0, semk).wait()


def _sc_attention(q, kv, routes):
    attn_fn = pl.kernel(
        _attn_body,
        mesh=plsc.VectorSubcoreMesh(core_axis_name="c", subcore_axis_name="s"),
        out_type=jax.ShapeDtypeStruct((SEQ, DIM), jnp.float32),
        scratch_types=[
            pltpu.VMEM((KNBR,), jnp.int32),
            pltpu.VMEM((KNBR,), jnp.int32),
            pltpu.VMEM((KNBR, DIM), jnp.int32),
            pltpu.VMEM((KNBR, DIM), jnp.int32),
            pltpu.VMEM((DIM,), jnp.float32),
            pltpu.VMEM((DIM,), jnp.float32),
            pltpu.VMEM((KNBR, 16), jnp.float32),
            pltpu.VMEM((DIM,), jnp.float32),
            pltpu.SemaphoreType.DMA,
        ],
    )
    return attn_fn(q, kv, routes)


def kernel(x, W_qkv, b_qkv, W_out, b_out, routes):
    xs = x.reshape(SEQ, DIM)
    rows = np.concatenate([PERM, DIM + PERMK, 2 * DIM + PERMK])
    scl = jnp.concatenate([
        jnp.full((DIM,), SCALE, jnp.float32),
        jnp.ones((2 * DIM,), jnp.float32)])
    wqkv = jnp.take(W_qkv, rows, axis=0) * scl[:, None]
    bqkv = jnp.take(b_qkv, rows) * scl

    q, kv = _matmul_qkv(xs, wqkv, bqkv)
    attn = _sc_attention(q, kv, routes)
    out = _matmul(attn, jnp.take(W_out, PERM, axis=1), b_out)
    return out.reshape(1, SEQ, DIM)


# speedup vs baseline: 1.2103x; 1.1161x over previous
import functools
import math

import jax
import jax.numpy as jnp
import numpy as np
from jax import lax
from jax.experimental import pallas as pl
from jax.experimental.pallas import tpu as pltpu
from jax.experimental.pallas import tpu_sc as plsc

SEQ = 2048
DIM = 1024
NUM_HEADS = 16
HEAD_DIM = 64
KNBR = 32
SCALE = 1.0 / math.sqrt(HEAD_DIM)

_J = np.arange(DIM)
PERM = np.asarray((_J % NUM_HEADS) * HEAD_DIM + _J // NUM_HEADS, dtype=np.int32)

_W = np.arange(DIM // 2)
_DPW = _W // 16
_HW = _W % 16
PERMK = np.concatenate([
    _HW * HEAD_DIM + 2 * _DPW,
    _HW * HEAD_DIM + 2 * _DPW + 1]).astype(np.int32)
_WPR = DIM // 2


def _mm_kernel(a_ref, w_ref, b_ref, o_ref):
    acc = lax.dot_general(
        a_ref[...], w_ref[...],
        dimension_numbers=(((1,), (1,)), ((), ())),
        preferred_element_type=jnp.float32,
    )
    o_ref[...] = (acc + b_ref[0, :][None, :]).astype(o_ref.dtype)


def _matmul(a, w, b, bm=512, bn=512, out_dtype=jnp.float32):
    a = a.astype(jnp.bfloat16)
    w = w.astype(jnp.bfloat16)
    m, k = a.shape
    n = w.shape[0]
    b2 = b.reshape(1, n)
    return pl.pallas_call(
        _mm_kernel,
        grid=(m // bm, n // bn),
        in_specs=[
            pl.BlockSpec((bm, k), lambda i, j: (i, 0)),
            pl.BlockSpec((bn, k), lambda i, j: (j, 0)),
            pl.BlockSpec((1, bn), lambda i, j: (0, j)),
        ],
        out_specs=pl.BlockSpec((bm, bn), lambda i, j: (i, j)),
        out_shape=jax.ShapeDtypeStruct((m, n), out_dtype),
    )(a, w, b2)


def _mm_pack_kernel(a_ref, w_ref, b_ref, o_ref):
    acc = lax.dot_general(
        a_ref[...], w_ref[...],
        dimension_numbers=(((1,), (1,)), ((), ())),
        preferred_element_type=jnp.float32,
    )
    acc = acc + b_ref[0, :][None, :]
    half = acc.shape[1] // 2
    lo = lax.bitcast_convert_type(
        acc[:, :half].astype(jnp.bfloat16), jnp.uint16).astype(jnp.int32)
    hi = lax.bitcast_convert_type(
        acc[:, half:].astype(jnp.bfloat16), jnp.uint16).astype(jnp.int32)
    o_ref[...] = jnp.bitwise_or(lo, lax.shift_left(hi, 16))


def _pack_half(sub):
    half = sub.shape[1] // 2
    lo = lax.bitcast_convert_type(
        sub[:, :half].astype(jnp.bfloat16), jnp.uint16).astype(jnp.int32)
    hi = lax.bitcast_convert_type(
        sub[:, half:].astype(jnp.bfloat16), jnp.uint16).astype(jnp.int32)
    return jnp.bitwise_or(lo, lax.shift_left(hi, 16))


def _mm_qkv_kernel(a_ref, w_ref, b_ref, q_ref, kv_ref):
    acc = lax.dot_general(
        a_ref[...], w_ref[...],
        dimension_numbers=(((1,), (1,)), ((), ())),
        preferred_element_type=jnp.float32,
    )
    acc = acc + b_ref[0, :][None, :]
    q_ref[...] = acc[:, :DIM]
    kv_ref[:, :DIM // 2] = _pack_half(acc[:, DIM:2 * DIM])
    kv_ref[:, DIM // 2:] = _pack_half(acc[:, 2 * DIM:])


def _matmul_qkv(a, w, b, bm=512):
    a = a.astype(jnp.bfloat16)
    w = w.astype(jnp.bfloat16)
    m, k = a.shape
    n = w.shape[0]
    b2 = b.reshape(1, n)
    return pl.pallas_call(
        _mm_qkv_kernel,
        grid=(m // bm,),
        in_specs=[
            pl.BlockSpec((bm, k), lambda i: (i, 0)),
            pl.BlockSpec((n, k), lambda i: (0, 0)),
            pl.BlockSpec((1, n), lambda i: (0, 0)),
        ],
        out_specs=[
            pl.BlockSpec((bm, DIM), lambda i: (i, 0)),
            pl.BlockSpec((bm, DIM), lambda i: (i, 0)),
        ],
        out_shape=[
            jax.ShapeDtypeStruct((m, DIM), jnp.float32),
            jax.ShapeDtypeStruct((m, DIM), jnp.int32),
        ],
    )(a, w, b2)


def _matmul_packed(a, w, b, bm=512):
    a = a.astype(jnp.bfloat16)
    w = w.astype(jnp.bfloat16)
    m, k = a.shape
    n = w.shape[0]
    b2 = b.reshape(1, n)
    return pl.pallas_call(
        _mm_pack_kernel,
        grid=(m // bm,),
        in_specs=[
            pl.BlockSpec((bm, k), lambda i: (i, 0)),
            pl.BlockSpec((n, k), lambda i: (0, 0)),
            pl.BlockSpec((1, n), lambda i: (0, 0)),
        ],
        out_specs=pl.BlockSpec((bm, n // 2), lambda i: (i, 0)),
        out_shape=jax.ShapeDtypeStruct((m, n // 2), jnp.int32),
    )(a, w, b2)


_NC, _NS = 2, 16
_NW = _NC * _NS
_QPW = SEQ // _NW
_JT = 8
_PT = 8
_NDP = HEAD_DIM // 2


def _bf16_pair(w):
    lo = lax.bitcast_convert_type(jnp.left_shift(w, 16), jnp.float32)
    hi = lax.bitcast_convert_type(w, jnp.float32)
    return lo, hi


def _attn_body(q_hbm, kv_hbm, routes_hbm, out_hbm,
               rbuf, kbuf0, kbuf1, qbuf0, qbuf1, sbuf, orow,
               semk):
    wid = lax.axis_index("s") * _NC + lax.axis_index("c")
    base = wid * _QPW

    kbufs = (kbuf0, kbuf1)
    qbufs = (qbuf0, qbuf1)

    pltpu.sync_copy(routes_hbm.at[pl.ds(base, _QPW)], rbuf)
    pltpu.async_copy(kv_hbm.at[rbuf.at[0]], kbuf0, semk)
    pltpu.async_copy(q_hbm.at[base], qbuf0, semk)

    def one_query(slot, li, linext):
        i = base + li
        kb, qb = kbufs[slot], qbufs[slot]
        kbn, qbn = kbufs[1 - slot], qbufs[1 - slot]

        pltpu.make_async_copy(kv_hbm.at[rbuf.at[li]], kb, semk).wait()
        pltpu.make_async_copy(q_hbm.at[i], qb, semk).wait()

        pltpu.async_copy(kv_hbm.at[rbuf.at[linext]], kbn, semk)
        pltpu.async_copy(q_hbm.at[base + linext], qbn, semk)

        m = jnp.full((16,), -jnp.inf, jnp.float32)
        for jt in range(KNBR // _JT):
            def dot_dp(dp, accs, jt=jt):
                q0 = qb[pl.ds(dp * 32, 16)]
                q1 = qb[pl.ds(dp * 32 + 16, 16)]
                out = []
                for u in range(_JT):
                    a, bb = _bf16_pair(kb[jt * _JT + u, pl.ds(dp * 16, 16)])
                    out.append(accs[u] + q0 * a + q1 * bb)
                return tuple(out)

            accs = lax.fori_loop(
                0, _NDP, dot_dp,
                tuple(jnp.zeros(16, jnp.float32) for _ in range(_JT)),
                unroll=2)
            for u in range(_JT):
                sc = accs[u]
                sbuf[jt * _JT + u, :] = sc
                m = jnp.maximum(m, sc)

        def exp_j(j, s):
            e = jnp.exp(sbuf[j, :] - m)
            sbuf[j, :] = e
            return s + e

        s = lax.fori_loop(0, KNBR, exp_j, jnp.zeros(16, jnp.float32), unroll=2)
        r = 1.0 / s

        for pt in range(_NDP // _PT):
            def acc_j(j, accs, pt=pt):
                wv = sbuf[j, :]
                out = list(accs)
                for t in range(_PT):
                    a, bb = _bf16_pair(
                        kb[j, pl.ds(_WPR + (pt * _PT + t) * 16, 16)])
                    out[2 * t] = out[2 * t] + wv * a
                    out[2 * t + 1] = out[2 * t + 1] + wv * bb
                return tuple(out)

            accs = lax.fori_loop(
                0, KNBR, acc_j,
                tuple(jnp.zeros(16, jnp.float32) for _ in range(2 * _PT)),
                unroll=2)
            for t in range(_PT):
                orow[pl.ds((pt * _PT + t) * 32, 16)] = accs[2 * t] * r
                orow[pl.ds((pt * _PT + t) * 32 + 16, 16)] = accs[2 * t + 1] * r

        pltpu.sync_copy(orow, out_hbm.at[i])

    def pair(h, _):
        li0 = 2 * h
        one_query(0, li0, li0 + 1)
        linext = jnp.where(li0 + 2 < _QPW, li0 + 2, 0)
        one_query(1, li0 + 1, linext)
        return 0

    lax.fori_loop(0, _QPW // 2, pair, 0)

    pltpu.make_async_copy(kv_hbm.at[rbuf.at[0]], kbuf0, semk).wait()
    pltpu.make_async_copy(q_hbm.at[base], qbuf0, semk).wait()


def _sc_attention(q, kv, routes):
    attn_fn = pl.kernel(
        _attn_body,
        mesh=plsc.VectorSubcoreMesh(core_axis_name="c", subcore_axis_name="s"),
        out_type=jax.ShapeDtypeStruct((SEQ, DIM), jnp.float32),
        scratch_types=[
            pltpu.VMEM((_QPW, KNBR), jnp.int32),
            pltpu.VMEM((KNBR, DIM), jnp.int32),
            pltpu.VMEM((KNBR, DIM), jnp.int32),
            pltpu.VMEM((DIM,), jnp.float32),
            pltpu.VMEM((DIM,), jnp.float32),
            pltpu.VMEM((KNBR, 16), jnp.float32),
            pltpu.VMEM((DIM,), jnp.float32),
            pltpu.SemaphoreType.DMA,
        ],
    )
    return attn_fn(q, kv, routes)


def kernel(x, W_qkv, b_qkv, W_out, b_out, routes):
    xs = x.reshape(SEQ, DIM)
    rows = np.concatenate([PERM, DIM + PERMK, 2 * DIM + PERMK])
    scl = jnp.concatenate([
        jnp.full((DIM,), SCALE, jnp.float32),
        jnp.ones((2 * DIM,), jnp.float32)])
    wqkv = jnp.take(W_qkv, rows, axis=0) * scl[:, None]
    bqkv = jnp.take(b_qkv, rows) * scl

    q, kv = _matmul_qkv(xs, wqkv, bqkv)
    attn = _sc_attention(q, kv, routes)
    out = _matmul(attn, jnp.take(W_out, PERM, axis=1), b_out)
    return out.reshape(1, SEQ, DIM)


# speedup vs baseline: 1.2551x; 1.0370x over previous
import functools
import math

import jax
import jax.numpy as jnp
import numpy as np
from jax import lax
from jax.experimental import pallas as pl
from jax.experimental.pallas import tpu as pltpu
from jax.experimental.pallas import tpu_sc as plsc

SEQ = 2048
DIM = 1024
NUM_HEADS = 16
HEAD_DIM = 64
KNBR = 32
SCALE = 1.0 / math.sqrt(HEAD_DIM)

_J = np.arange(DIM)
PERM = np.asarray((_J % NUM_HEADS) * HEAD_DIM + _J // NUM_HEADS, dtype=np.int32)

_W = np.arange(DIM // 2)
_DPW = _W // 16
_HW = _W % 16
PERMK = np.concatenate([
    _HW * HEAD_DIM + 2 * _DPW,
    _HW * HEAD_DIM + 2 * _DPW + 1]).astype(np.int32)
_WPR = DIM // 2


def _mm_kernel(a_ref, w_ref, b_ref, o_ref):
    acc = lax.dot_general(
        a_ref[...], w_ref[...],
        dimension_numbers=(((1,), (1,)), ((), ())),
        preferred_element_type=jnp.float32,
    )
    o_ref[...] = (acc + b_ref[0, :][None, :]).astype(o_ref.dtype)


def _matmul(a, w, b, bm=512, bn=512, out_dtype=jnp.float32):
    a = a.astype(jnp.bfloat16)
    w = w.astype(jnp.bfloat16)
    m, k = a.shape
    n = w.shape[0]
    b2 = b.reshape(1, n)
    return pl.pallas_call(
        _mm_kernel,
        grid=(m // bm, n // bn),
        in_specs=[
            pl.BlockSpec((bm, k), lambda i, j: (i, 0)),
            pl.BlockSpec((bn, k), lambda i, j: (j, 0)),
            pl.BlockSpec((1, bn), lambda i, j: (0, j)),
        ],
        out_specs=pl.BlockSpec((bm, bn), lambda i, j: (i, j)),
        out_shape=jax.ShapeDtypeStruct((m, n), out_dtype),
    )(a, w, b2)


def _mm_pack_kernel(a_ref, w_ref, b_ref, o_ref):
    acc = lax.dot_general(
        a_ref[...], w_ref[...],
        dimension_numbers=(((1,), (1,)), ((), ())),
        preferred_element_type=jnp.float32,
    )
    acc = acc + b_ref[0, :][None, :]
    half = acc.shape[1] // 2
    lo = lax.bitcast_convert_type(
        acc[:, :half].astype(jnp.bfloat16), jnp.uint16).astype(jnp.int32)
    hi = lax.bitcast_convert_type(
        acc[:, half:].astype(jnp.bfloat16), jnp.uint16).astype(jnp.int32)
    o_ref[...] = jnp.bitwise_or(lo, lax.shift_left(hi, 16))


def _pack_half(sub):
    half = sub.shape[1] // 2
    lo = lax.bitcast_convert_type(
        sub[:, :half].astype(jnp.bfloat16), jnp.uint16).astype(jnp.int32)
    hi = lax.bitcast_convert_type(
        sub[:, half:].astype(jnp.bfloat16), jnp.uint16).astype(jnp.int32)
    return jnp.bitwise_or(lo, lax.shift_left(hi, 16))


def _mm_qkv_kernel(a_ref, w_ref, b_ref, q_ref, kv_ref):
    acc = lax.dot_general(
        a_ref[...], w_ref[...],
        dimension_numbers=(((1,), (1,)), ((), ())),
        preferred_element_type=jnp.float32,
    )
    acc = acc + b_ref[0, :][None, :]
    q_ref[...] = acc[:, :DIM]
    kv_ref[:, :DIM // 2] = _pack_half(acc[:, DIM:2 * DIM])
    kv_ref[:, DIM // 2:] = _pack_half(acc[:, 2 * DIM:])


def _matmul_qkv(a, w, b, bm=512):
    a = a.astype(jnp.bfloat16)
    w = w.astype(jnp.bfloat16)
    m, k = a.shape
    n = w.shape[0]
    b2 = b.reshape(1, n)
    return pl.pallas_call(
        _mm_qkv_kernel,
        grid=(m // bm,),
        in_specs=[
            pl.BlockSpec((bm, k), lambda i: (i, 0)),
            pl.BlockSpec((n, k), lambda i: (0, 0)),
            pl.BlockSpec((1, n), lambda i: (0, 0)),
        ],
        out_specs=[
            pl.BlockSpec((bm, DIM), lambda i: (i, 0)),
            pl.BlockSpec((bm, DIM), lambda i: (i, 0)),
        ],
        out_shape=[
            jax.ShapeDtypeStruct((m, DIM), jnp.float32),
            jax.ShapeDtypeStruct((m, DIM), jnp.int32),
        ],
    )(a, w, b2)


def _matmul_packed(a, w, b, bm=512):
    a = a.astype(jnp.bfloat16)
    w = w.astype(jnp.bfloat16)
    m, k = a.shape
    n = w.shape[0]
    b2 = b.reshape(1, n)
    return pl.pallas_call(
        _mm_pack_kernel,
        grid=(m // bm,),
        in_specs=[
            pl.BlockSpec((bm, k), lambda i: (i, 0)),
            pl.BlockSpec((n, k), lambda i: (0, 0)),
            pl.BlockSpec((1, n), lambda i: (0, 0)),
        ],
        out_specs=pl.BlockSpec((bm, n // 2), lambda i: (i, 0)),
        out_shape=jax.ShapeDtypeStruct((m, n // 2), jnp.int32),
    )(a, w, b2)


_NC, _NS = 2, 16
_NW = _NC * _NS
_QPW = SEQ // _NW
_JT = 8
_PT = 8
_NDP = HEAD_DIM // 2


def _bf16_pair(w):
    lo = lax.bitcast_convert_type(jnp.left_shift(w, 16), jnp.float32)
    hi = lax.bitcast_convert_type(w, jnp.float32)
    return lo, hi


def _attn_body(q_hbm, kv_hbm, routes_hbm, out_hbm,
               rbuf, kbuf0, kbuf1, qbuf0, qbuf1, sbuf, orow0, orow1,
               semk, semo):
    wid = lax.axis_index("s") * _NC + lax.axis_index("c")
    base = wid * _QPW

    kbufs = (kbuf0, kbuf1)
    qbufs = (qbuf0, qbuf1)
    orows = (orow0, orow1)

    pltpu.sync_copy(routes_hbm.at[pl.ds(base, _QPW)], rbuf)
    pltpu.async_copy(kv_hbm.at[rbuf.at[0]], kbuf0, semk)
    pltpu.async_copy(q_hbm.at[base], qbuf0, semk)
    pltpu.async_copy(orow0, out_hbm.at[base], semo)
    pltpu.async_copy(orow1, out_hbm.at[base + 1], semo)

    def one_query(slot, li, linext):
        i = base + li
        kb, qb = kbufs[slot], qbufs[slot]
        kbn, qbn = kbufs[1 - slot], qbufs[1 - slot]
        orow = orows[slot]

        pltpu.make_async_copy(kv_hbm.at[rbuf.at[li]], kb, semk).wait()
        pltpu.make_async_copy(q_hbm.at[i], qb, semk).wait()

        pltpu.async_copy(kv_hbm.at[rbuf.at[linext]], kbn, semk)
        pltpu.async_copy(q_hbm.at[base + linext], qbn, semk)

        m = jnp.full((16,), -jnp.inf, jnp.float32)
        for jt in range(KNBR // _JT):
            def dot_dp(dp, accs, jt=jt):
                q0 = qb[pl.ds(dp * 32, 16)]
                q1 = qb[pl.ds(dp * 32 + 16, 16)]
                out = []
                for u in range(_JT):
                    a, bb = _bf16_pair(kb[jt * _JT + u, pl.ds(dp * 16, 16)])
                    out.append(accs[u] + q0 * a + q1 * bb)
                return tuple(out)

            accs = lax.fori_loop(
                0, _NDP, dot_dp,
                tuple(jnp.zeros(16, jnp.float32) for _ in range(_JT)),
                unroll=2)
            for u in range(_JT):
                sc = accs[u]
                sbuf[jt * _JT + u, :] = sc
                m = jnp.maximum(m, sc)

        def exp_j(j, s):
            e = jnp.exp(sbuf[j, :] - m)
            sbuf[j, :] = e
            return s + e

        s = lax.fori_loop(0, KNBR, exp_j, jnp.zeros(16, jnp.float32), unroll=2)
        r = 1.0 / s

        pltpu.make_async_copy(orow, out_hbm.at[i], semo).wait()

        for pt in range(_NDP // _PT):
            def acc_j(j, accs, pt=pt):
                wv = sbuf[j, :]
                out = list(accs)
                for t in range(_PT):
                    a, bb = _bf16_pair(
                        kb[j, pl.ds(_WPR + (pt * _PT + t) * 16, 16)])
                    out[2 * t] = out[2 * t] + wv * a
                    out[2 * t + 1] = out[2 * t + 1] + wv * bb
                return tuple(out)

            accs = lax.fori_loop(
                0, KNBR, acc_j,
                tuple(jnp.zeros(16, jnp.float32) for _ in range(2 * _PT)),
                unroll=2)
            for t in range(_PT):
                orow[pl.ds((pt * _PT + t) * 32, 16)] = accs[2 * t] * r
                orow[pl.ds((pt * _PT + t) * 32 + 16, 16)] = accs[2 * t + 1] * r

        pltpu.async_copy(orow, out_hbm.at[i], semo)

    def pair(h, _):
        li0 = 2 * h
        one_query(0, li0, li0 + 1)
        linext = jnp.where(li0 + 2 < _QPW, li0 + 2, 0)
        one_query(1, li0 + 1, linext)
        return 0

    lax.fori_loop(0, _QPW // 2, pair, 0)

    pltpu.make_async_copy(kv_hbm.at[rbuf.at[0]], kbuf0, semk).wait()
    pltpu.make_async_copy(q_hbm.at[base], qbuf0, semk).wait()
    pltpu.make_async_copy(orow0, out_hbm.at[base], semo).wait()
    pltpu.make_async_copy(orow1, out_hbm.at[base + 1], semo).wait()


def _sc_attention(q, kv, routes):
    attn_fn = pl.kernel(
        _attn_body,
        mesh=plsc.VectorSubcoreMesh(core_axis_name="c", subcore_axis_name="s"),
        out_type=jax.ShapeDtypeStruct((SEQ, DIM), jnp.float32),
        scratch_types=[
            pltpu.VMEM((_QPW, KNBR), jnp.int32),
            pltpu.VMEM((KNBR, DIM), jnp.int32),
            pltpu.VMEM((KNBR, DIM), jnp.int32),
            pltpu.VMEM((DIM,), jnp.float32),
            pltpu.VMEM((DIM,), jnp.float32),
            pltpu.VMEM((KNBR, 16), jnp.float32),
            pltpu.VMEM((DIM,), jnp.float32),
            pltpu.VMEM((DIM,), jnp.float32),
            pltpu.SemaphoreType.DMA,
            pltpu.SemaphoreType.DMA,
        ],
    )
    return attn_fn(q, kv, routes)


def kernel(x, W_qkv, b_qkv, W_out, b_out, routes):
    xs = x.reshape(SEQ, DIM)
    rows = np.concatenate([PERM, DIM + PERMK, 2 * DIM + PERMK])
    scl = jnp.concatenate([
        jnp.full((DIM,), SCALE, jnp.float32),
        jnp.ones((2 * DIM,), jnp.float32)])
    wqkv = jnp.take(W_qkv, rows, axis=0) * scl[:, None]
    bqkv = jnp.take(b_qkv, rows) * scl

    q, kv = _matmul_qkv(xs, wqkv, bqkv)
    attn = _sc_attention(q, kv, routes)
    out = _matmul(attn, jnp.take(W_out, PERM, axis=1), b_out)
    return out.reshape(1, SEQ, DIM)


# speedup vs baseline: 1.2962x; 1.0327x over previous
import functools
import math

import jax
import jax.numpy as jnp
import numpy as np
from jax import lax
from jax.experimental import pallas as pl
from jax.experimental.pallas import tpu as pltpu
from jax.experimental.pallas import tpu_sc as plsc

SEQ = 2048
DIM = 1024
NUM_HEADS = 16
HEAD_DIM = 64
KNBR = 32
SCALE = 1.0 / math.sqrt(HEAD_DIM)

_J = np.arange(DIM)
PERM = np.asarray((_J % NUM_HEADS) * HEAD_DIM + _J // NUM_HEADS, dtype=np.int32)

_W = np.arange(DIM // 2)
_DPW = _W // 16
_HW = _W % 16
PERMK = np.concatenate([
    _HW * HEAD_DIM + 2 * _DPW,
    _HW * HEAD_DIM + 2 * _DPW + 1]).astype(np.int32)
_WPR = DIM // 2


def _mm_kernel(a_ref, w_ref, b_ref, o_ref):
    acc = lax.dot_general(
        a_ref[...], w_ref[...],
        dimension_numbers=(((1,), (1,)), ((), ())),
        preferred_element_type=jnp.float32,
    )
    o_ref[...] = (acc + b_ref[0, :][None, :]).astype(o_ref.dtype)


def _matmul(a, w, b, bm=512, bn=512, out_dtype=jnp.float32):
    a = a.astype(jnp.bfloat16)
    w = w.astype(jnp.bfloat16)
    m, k = a.shape
    n = w.shape[0]
    b2 = b.reshape(1, n)
    return pl.pallas_call(
        _mm_kernel,
        grid=(m // bm, n // bn),
        in_specs=[
            pl.BlockSpec((bm, k), lambda i, j: (i, 0)),
            pl.BlockSpec((bn, k), lambda i, j: (j, 0)),
            pl.BlockSpec((1, bn), lambda i, j: (0, j)),
        ],
        out_specs=pl.BlockSpec((bm, bn), lambda i, j: (i, j)),
        out_shape=jax.ShapeDtypeStruct((m, n), out_dtype),
    )(a, w, b2)


def _mm_pack_kernel(a_ref, w_ref, b_ref, o_ref):
    acc = lax.dot_general(
        a_ref[...], w_ref[...],
        dimension_numbers=(((1,), (1,)), ((), ())),
        preferred_element_type=jnp.float32,
    )
    acc = acc + b_ref[0, :][None, :]
    half = acc.shape[1] // 2
    lo = lax.bitcast_convert_type(
        acc[:, :half].astype(jnp.bfloat16), jnp.uint16).astype(jnp.int32)
    hi = lax.bitcast_convert_type(
        acc[:, half:].astype(jnp.bfloat16), jnp.uint16).astype(jnp.int32)
    o_ref[...] = jnp.bitwise_or(lo, lax.shift_left(hi, 16))


def _pack_half(sub):
    half = sub.shape[1] // 2
    lo = lax.bitcast_convert_type(
        sub[:, :half].astype(jnp.bfloat16), jnp.uint16).astype(jnp.int32)
    hi = lax.bitcast_convert_type(
        sub[:, half:].astype(jnp.bfloat16), jnp.uint16).astype(jnp.int32)
    return jnp.bitwise_or(lo, lax.shift_left(hi, 16))


def _mm_qkv_kernel(a_ref, w_ref, b_ref, q_ref, kv_ref):
    acc = lax.dot_general(
        a_ref[...], w_ref[...],
        dimension_numbers=(((1,), (1,)), ((), ())),
        preferred_element_type=jnp.float32,
    )
    acc = acc + b_ref[0, :][None, :]
    q_ref[...] = acc[:, :DIM]
    kv_ref[:, :DIM // 2] = _pack_half(acc[:, DIM:2 * DIM])
    kv_ref[:, DIM // 2:] = _pack_half(acc[:, 2 * DIM:])


def _matmul_qkv(a, w, b, bm=512):
    a = a.astype(jnp.bfloat16)
    w = w.astype(jnp.bfloat16)
    m, k = a.shape
    n = w.shape[0]
    b2 = b.reshape(1, n)
    return pl.pallas_call(
        _mm_qkv_kernel,
        grid=(m // bm,),
        in_specs=[
            pl.BlockSpec((bm, k), lambda i: (i, 0)),
            pl.BlockSpec((n, k), lambda i: (0, 0)),
            pl.BlockSpec((1, n), lambda i: (0, 0)),
        ],
        out_specs=[
            pl.BlockSpec((bm, DIM), lambda i: (i, 0)),
            pl.BlockSpec((bm, DIM), lambda i: (i, 0)),
        ],
        out_shape=[
            jax.ShapeDtypeStruct((m, DIM), jnp.float32),
            jax.ShapeDtypeStruct((m, DIM), jnp.int32),
        ],
    )(a, w, b2)


def _matmul_packed(a, w, b, bm=512):
    a = a.astype(jnp.bfloat16)
    w = w.astype(jnp.bfloat16)
    m, k = a.shape
    n = w.shape[0]
    b2 = b.reshape(1, n)
    return pl.pallas_call(
        _mm_pack_kernel,
        grid=(m // bm,),
        in_specs=[
            pl.BlockSpec((bm, k), lambda i: (i, 0)),
            pl.BlockSpec((n, k), lambda i: (0, 0)),
            pl.BlockSpec((1, n), lambda i: (0, 0)),
        ],
        out_specs=pl.BlockSpec((bm, n // 2), lambda i: (i, 0)),
        out_shape=jax.ShapeDtypeStruct((m, n // 2), jnp.int32),
    )(a, w, b2)


_NC, _NS = 2, 16
_NW = _NC * _NS
_QPW = SEQ // _NW
_JT = 8
_PT = 8
_NDP = HEAD_DIM // 2


def _bf16_pair(w):
    lo = lax.bitcast_convert_type(jnp.left_shift(w, 16), jnp.float32)
    hi = lax.bitcast_convert_type(w, jnp.float32)
    return lo, hi


def _attn_body(q_hbm, kv_hbm, routes_hbm, out_hbm,
               rbuf, kbuf0, kbuf1, qbuf0, qbuf1, sbuf, orow0, orow1,
               semk, semo):
    wid = lax.axis_index("s") * _NC + lax.axis_index("c")
    base = wid * _QPW

    kbufs = (kbuf0, kbuf1)
    qbufs = (qbuf0, qbuf1)
    orows = (orow0, orow1)

    pltpu.sync_copy(routes_hbm.at[pl.ds(base, _QPW)], rbuf)
    pltpu.async_copy(kv_hbm.at[rbuf.at[0]], kbuf0, semk)
    pltpu.async_copy(q_hbm.at[base], qbuf0, semk)
    pltpu.async_copy(orow0, out_hbm.at[base], semo)
    pltpu.async_copy(orow1, out_hbm.at[base + 1], semo)

    def one_query(slot, li, linext):
        i = base + li
        kb, qb = kbufs[slot], qbufs[slot]
        kbn, qbn = kbufs[1 - slot], qbufs[1 - slot]
        orow = orows[slot]

        pltpu.make_async_copy(kv_hbm.at[rbuf.at[li]], kb, semk).wait()
        pltpu.make_async_copy(q_hbm.at[i], qb, semk).wait()

        pltpu.async_copy(kv_hbm.at[rbuf.at[linext]], kbn, semk)
        pltpu.async_copy(q_hbm.at[base + linext], qbn, semk)

        m = jnp.full((16,), -jnp.inf, jnp.float32)
        for jt in range(KNBR // _JT):
            def dot_dp(dp, accs, jt=jt):
                q0 = qb[pl.ds(dp * 32, 16)]
                q1 = qb[pl.ds(dp * 32 + 16, 16)]
                out = []
                for u in range(_JT):
                    a, bb = _bf16_pair(kb[jt * _JT + u, pl.ds(dp * 16, 16)])
                    out.append(accs[u] + q0 * a + q1 * bb)
                return tuple(out)

            accs = lax.fori_loop(
                0, _NDP, dot_dp,
                tuple(jnp.zeros(16, jnp.float32) for _ in range(_JT)),
                unroll=2)
            for u in range(_JT):
                sc = accs[u]
                sbuf[jt * _JT + u, :] = sc
                m = jnp.maximum(m, sc)

        pltpu.make_async_copy(orow, out_hbm.at[i], semo).wait()

        def acc_j0(j, carry):
            sa = carry[0]
            e = jnp.exp(sbuf[j, :] - m)
            sbuf[j, :] = e
            out = list(carry[1:])
            for t in range(_PT):
                a, bb = _bf16_pair(kb[j, pl.ds(_WPR + t * 16, 16)])
                out[2 * t] = out[2 * t] + e * a
                out[2 * t + 1] = out[2 * t + 1] + e * bb
            return (sa + e,) + tuple(out)

        carry = lax.fori_loop(
            0, KNBR, acc_j0,
            tuple(jnp.zeros(16, jnp.float32) for _ in range(2 * _PT + 1)),
            unroll=2)
        r = 1.0 / carry[0]
        for t in range(_PT):
            orow[pl.ds(t * 32, 16)] = carry[1 + 2 * t] * r
            orow[pl.ds(t * 32 + 16, 16)] = carry[2 + 2 * t] * r

        for pt in range(1, _NDP // _PT):
            def acc_j(j, accs, pt=pt):
                wv = sbuf[j, :]
                out = list(accs)
                for t in range(_PT):
                    a, bb = _bf16_pair(
                        kb[j, pl.ds(_WPR + (pt * _PT + t) * 16, 16)])
                    out[2 * t] = out[2 * t] + wv * a
                    out[2 * t + 1] = out[2 * t + 1] + wv * bb
                return tuple(out)

            accs = lax.fori_loop(
                0, KNBR, acc_j,
                tuple(jnp.zeros(16, jnp.float32) for _ in range(2 * _PT)),
                unroll=2)
            for t in range(_PT):
                orow[pl.ds((pt * _PT + t) * 32, 16)] = accs[2 * t] * r
                orow[pl.ds((pt * _PT + t) * 32 + 16, 16)] = accs[2 * t + 1] * r

        pltpu.async_copy(orow, out_hbm.at[i], semo)

    def pair(h, _):
        li0 = 2 * h
        one_query(0, li0, li0 + 1)
        linext = jnp.where(li0 + 2 < _QPW, li0 + 2, 0)
        one_query(1, li0 + 1, linext)
        return 0

    lax.fori_loop(0, _QPW // 2, pair, 0)

    pltpu.make_async_copy(kv_hbm.at[rbuf.at[0]], kbuf0, semk).wait()
    pltpu.make_async_copy(q_hbm.at[base], qbuf0, semk).wait()
    pltpu.make_async_copy(orow0, out_hbm.at[base], semo).wait()
    pltpu.make_async_copy(orow1, out_hbm.at[base + 1], semo).wait()


def _sc_attention(q, kv, routes):
    attn_fn = pl.kernel(
        _attn_body,
        mesh=plsc.VectorSubcoreMesh(core_axis_name="c", subcore_axis_name="s"),
        out_type=jax.ShapeDtypeStruct((SEQ, DIM), jnp.float32),
        scratch_types=[
            pltpu.VMEM((_QPW, KNBR), jnp.int32),
            pltpu.VMEM((KNBR, DIM), jnp.int32),
            pltpu.VMEM((KNBR, DIM), jnp.int32),
            pltpu.VMEM((DIM,), jnp.float32),
            pltpu.VMEM((DIM,), jnp.float32),
            pltpu.VMEM((KNBR, 16), jnp.float32),
            pltpu.VMEM((DIM,), jnp.float32),
            pltpu.VMEM((DIM,), jnp.float32),
            pltpu.SemaphoreType.DMA,
            pltpu.SemaphoreType.DMA,
        ],
    )
    return attn_fn(q, kv, routes)


def kernel(x, W_qkv, b_qkv, W_out, b_out, routes):
    xs = x.reshape(SEQ, DIM)
    rows = np.concatenate([PERM, DIM + PERMK, 2 * DIM + PERMK])
    scl = jnp.concatenate([
        jnp.full((DIM,), SCALE, jnp.float32),
        jnp.ones((2 * DIM,), jnp.float32)])
    wqkv = jnp.take(W_qkv, rows, axis=0) * scl[:, None]
    bqkv = jnp.take(b_qkv, rows) * scl

    q, kv = _matmul_qkv(xs, wqkv, bqkv)
    attn = _sc_attention(q, kv, routes)
    out = _matmul(attn, jnp.take(W_out, PERM, axis=1), b_out)
    return out.reshape(1, SEQ, DIM)
